# baseline (speedup 1.0000x reference)
import functools

import jax
import jax.numpy as jnp
from jax.experimental import pallas as pl
from jax.experimental.pallas import tpu as pltpu

VOCAB_ = 100000
HID_ = 128
TILE_ = 4096
NTILES_ = 25
PADV_ = NTILES_ * TILE_


def _decoder_body(idx_ref, emb_ref, w_ref, b_ref, out_ref, logits_ref, acc_ref):
    i = pl.program_id(0)

    @pl.when(i == 0)
    def _init():
        acc_ref[0] = -1e30
        acc_ref[1] = 0.0

    x = jnp.maximum(emb_ref[0], 0.0)
    t = jnp.dot(x, w_ref[...], preferred_element_type=jnp.float32) + b_ref[...]
    col = i * TILE_ + jax.lax.broadcasted_iota(jnp.int32, (1, TILE_), 1)
    t = jnp.where(col < VOCAB_, t, -1e30)
    logits_ref[:, pl.ds(i * TILE_, TILE_)] = t

    m_old = acc_ref[0]
    m_new = jnp.maximum(m_old, jnp.max(t))
    s_new = acc_ref[1] * jnp.exp(m_old - m_new) + jnp.sum(jnp.exp(t - m_new))
    acc_ref[0] = m_new
    acc_ref[1] = s_new

    @pl.when(i == NTILES_ - 1)
    def _epilogue():
        out_ref[...] = logits_ref[...] - (acc_ref[0] + jnp.log(acc_ref[1]))


@functools.partial(jax.jit, static_argnames=("interpret",))
def kernel(input, table, W, b, interpret=False):
    b2 = b.reshape(1, VOCAB_)
    table3 = table.reshape(VOCAB_, 1, HID_)
    grid_spec = pltpu.PrefetchScalarGridSpec(
        num_scalar_prefetch=1,
        grid=(NTILES_,),
        in_specs=[
            pl.BlockSpec((1, 1, HID_), lambda i, idx_ref: (idx_ref[0], 0, 0)),
            pl.BlockSpec((HID_, TILE_), lambda i, idx_ref: (0, i)),
            pl.BlockSpec((1, TILE_), lambda i, idx_ref: (0, i)),
        ],
        out_specs=pl.BlockSpec((1, PADV_), lambda i, idx_ref: (0, 0)),
        scratch_shapes=[
            pltpu.VMEM((1, PADV_), jnp.float32),
            pltpu.SMEM((2,), jnp.float32),
        ],
    )
    out = pl.pallas_call(
        _decoder_body,
        grid_spec=grid_spec,
        out_shape=jax.ShapeDtypeStruct((1, PADV_), jnp.float32),
        interpret=interpret,
    )(input, table3, W, b2)
    return out[:, :VOCAB_]


# speedup vs baseline: 1.1158x; 1.1158x over previous
import functools

import jax
import jax.numpy as jnp
from jax.experimental import pallas as pl
from jax.experimental.pallas import tpu as pltpu

VOCAB_ = 100000
HID_ = 128
TILE_ = 8192
NTILES_ = 13
PADV_ = NTILES_ * TILE_


def _decoder_body(idx_ref, emb_ref, w_ref, b_ref, out_ref, logits_ref, acc_ref):
    i = pl.program_id(0)

    @pl.when(i == 0)
    def _init():
        acc_ref[0] = -1e30
        acc_ref[1] = 0.0

    x = jnp.maximum(emb_ref[0], 0.0)
    t = jnp.dot(x, w_ref[...], preferred_element_type=jnp.float32) + b_ref[...]
    col = i * TILE_ + jax.lax.broadcasted_iota(jnp.int32, (1, TILE_), 1)
    t = jnp.where(col < VOCAB_, t, -1e30)
    logits_ref[:, pl.ds(i * TILE_, TILE_)] = t

    m_old = acc_ref[0]
    m_new = jnp.maximum(m_old, jnp.max(t))
    s_new = acc_ref[1] * jnp.exp(m_old - m_new) + jnp.sum(jnp.exp(t - m_new))
    acc_ref[0] = m_new
    acc_ref[1] = s_new

    @pl.when(i == NTILES_ - 1)
    def _epilogue():
        out_ref[...] = logits_ref[...] - (acc_ref[0] + jnp.log(acc_ref[1]))


@functools.partial(jax.jit, static_argnames=("interpret",))
def kernel(input, table, W, b, interpret=False):
    b2 = b.reshape(1, VOCAB_)
    table3 = table.reshape(VOCAB_, 1, HID_)
    grid_spec = pltpu.PrefetchScalarGridSpec(
        num_scalar_prefetch=1,
        grid=(NTILES_,),
        in_specs=[
            pl.BlockSpec((1, 1, HID_), lambda i, idx_ref: (idx_ref[0], 0, 0)),
            pl.BlockSpec((HID_, TILE_), lambda i, idx_ref: (0, i)),
            pl.BlockSpec((1, TILE_), lambda i, idx_ref: (0, i)),
        ],
        out_specs=pl.BlockSpec((1, PADV_), lambda i, idx_ref: (0, 0)),
        scratch_shapes=[
            pltpu.VMEM((1, PADV_), jnp.float32),
            pltpu.SMEM((2,), jnp.float32),
        ],
    )
    out = pl.pallas_call(
        _decoder_body,
        grid_spec=grid_spec,
        out_shape=jax.ShapeDtypeStruct((1, PADV_), jnp.float32),
        interpret=interpret,
    )(input, table3, W, b2)
    return out[:, :VOCAB_]
